# Initial kernel scaffold; baseline (speedup 1.0000x reference)
import jax
import jax.numpy as jnp
from jax.experimental import pallas as pl


def kernel(neighbors, train_hedges, labels, neighborhedges, hyperedges, hedgetypes, W1, b1, W2, b2):
    raise NotImplementedError("write your pallas kernel here")


# speedup vs baseline: 15.7191x; 15.7191x over previous
import jax
import jax.numpy as jnp
from jax import lax
from jax.experimental import pallas as pl
from jax.experimental.pallas import tpu as pltpu
from jax.experimental.pallas import tpu_sc as plsc

B = 1024
NT = 128
HD = 512

NW = 32
BW = B // NW


def _sc_body(nbf_hbm, nh0, nh1, nh2, nh3, he0, he1, he2, he3, ht_hbm,
             h1_out, t1_out, h2_out, t2_out,
             nbf_v, h1_v, t1_v, n2_v, h2_v, t2_v, sem):
    wid = lax.axis_index("c") * 16 + lax.axis_index("s")
    nh_c = (nh0, nh1, nh2, nh3)
    he_c = (he0, he1, he2, he3)

    pltpu.sync_copy(nbf_hbm.at[pl.ds(wid * 128, 128)], nbf_v)

    cps = [pltpu.async_copy(nh_c[n].at[nbf_v], h1_v.at[n], sem)
           for n in range(4)]
    for cp in cps:
        cp.wait()

    cps = [pltpu.async_copy(ht_hbm.at[h1_v.at[n]], t1_v.at[n], sem)
           for n in range(4)]
    cps += [pltpu.async_copy(he_c[s].at[h1_v.at[n]], n2_v.at[n * 4 + s], sem)
            for n in range(4) for s in range(4)]
    for cp in cps:
        cp.wait()

    def batch_body(c, carry):
        cps = [pltpu.async_copy(nh_c[n2].at[n2_v.at[c]],
                                h2_v.at[4 * c + n2], sem)
               for n2 in range(4)]
        for cp in cps:
            cp.wait()
        cps = [pltpu.async_copy(ht_hbm.at[h2_v.at[4 * c + n2]],
                                t2_v.at[4 * c + n2], sem)
               for n2 in range(4)]
        for cp in cps:
            cp.wait()
        return carry

    lax.fori_loop(0, 16, batch_body, 0)

    pltpu.sync_copy(h1_v, h1_out.at[pl.ds(wid * 4, 4)])
    pltpu.sync_copy(t1_v, t1_out.at[pl.ds(wid * 4, 4)])
    pltpu.sync_copy(h2_v, h2_out.at[pl.ds(wid * 64, 64)])
    pltpu.sync_copy(t2_v, t2_out.at[pl.ds(wid * 64, 64)])


def _sc_gather(nbf, nh_cols, he_cols, hedgetypes):
    mesh = plsc.VectorSubcoreMesh(core_axis_name="c", subcore_axis_name="s")
    kfn = pl.kernel(
        _sc_body,
        mesh=mesh,
        compiler_params=pltpu.CompilerParams(
            needs_layout_passes=False,
            use_tc_tiling_on_sc=False,
        ),
        out_type=(
            jax.ShapeDtypeStruct((NW * 4, 128), jnp.int32),
            jax.ShapeDtypeStruct((NW * 4, 128), jnp.int32),
            jax.ShapeDtypeStruct((NW * 64, 128), jnp.int32),
            jax.ShapeDtypeStruct((NW * 64, 128), jnp.int32),
        ),
        scratch_types=[
            pltpu.VMEM((128,), jnp.int32),
            pltpu.VMEM((4, 128), jnp.int32),
            pltpu.VMEM((4, 128), jnp.int32),
            pltpu.VMEM((16, 128), jnp.int32),
            pltpu.VMEM((64, 128), jnp.int32),
            pltpu.VMEM((64, 128), jnp.int32),
            pltpu.SemaphoreType.DMA,
        ],
    )
    return kfn(nbf, *nh_cols, *he_cols, hedgetypes)


G = 4
R = G * BW * 16


def _tc_body(t1_ref, h1_ref, th_ref, t2_ref, h2_ref,
             W1_ref, b1_ref, W2_ref, b2_ref, out_ref):
    f32 = jnp.float32
    t2 = t2_ref[...].reshape(G, 4, 16, 128).transpose(0, 1, 3, 2).reshape(R, 16)
    h2 = h2_ref[...].reshape(G, 4, 16, 128).transpose(0, 1, 3, 2).reshape(R, 16)
    t1 = t1_ref[...]
    h1 = h1_ref[...]
    th = th_ref[...]

    iota = lax.broadcasted_iota(jnp.int32, (1, NT), 1)
    A = (t1 == iota).astype(f32)
    m2 = (h2 != th).astype(f32) * (1.0 / 16.0)
    for n in range(16):
        A = A + m2[:, n:n + 1] * (t2[:, n:n + 1] == iota).astype(f32)
    H = jnp.maximum(
        jnp.dot(A, W1_ref[...], preferred_element_type=f32) + b1_ref[...], 0.0)
    m1 = (h1 != th).astype(f32) * (1.0 / 16.0)
    P = H * m1
    Ph = P.reshape(R // 4, 4, HD).sum(axis=1)
    u = Ph.reshape(G, 4, BW, HD).sum(axis=1).reshape(G * BW, HD)
    out_ref[...] = jax.nn.sigmoid(
        jnp.dot(u, W2_ref[...], preferred_element_type=f32) + b2_ref[...])


def _tc_dense(t1c, h1c, th2d, t2c, h2c, W1, b1r, W2, b2r):
    nblk = NW // G
    return pl.pallas_call(
        _tc_body,
        grid=(nblk,),
        in_specs=[
            pl.BlockSpec((R, 1), lambda i: (i, 0)),
            pl.BlockSpec((R, 1), lambda i: (i, 0)),
            pl.BlockSpec((R, 1), lambda i: (i, 0)),
            pl.BlockSpec((G * 64, 128), lambda i: (i, 0)),
            pl.BlockSpec((G * 64, 128), lambda i: (i, 0)),
            pl.BlockSpec((NT, HD), lambda i: (0, 0)),
            pl.BlockSpec((1, HD), lambda i: (0, 0)),
            pl.BlockSpec((HD, NT), lambda i: (0, 0)),
            pl.BlockSpec((1, NT), lambda i: (0, 0)),
        ],
        out_specs=pl.BlockSpec((G * BW, NT), lambda i: (i, 0)),
        out_shape=jax.ShapeDtypeStruct((B, NT), jnp.float32),
    )(t1c, h1c, th2d, t2c, h2c, W1, b1r, W2, b2r)


def kernel(neighbors, train_hedges, labels, neighborhedges, hyperedges,
           hedgetypes, W1, b1, W2, b2):
    del labels
    nbf = neighbors.reshape(B * 4)
    nh_cols = tuple(neighborhedges[:, n] for n in range(4))
    he_cols = tuple(hyperedges[:, s] for s in range(4))
    h1c, t1c, h2c, t2c = _sc_gather(nbf, nh_cols, he_cols, hedgetypes)
    t1r = t1c.reshape(B * 16, 1)
    h1r = h1c.reshape(B * 16, 1)
    thr = jnp.broadcast_to(train_hedges.reshape(NW, 1, BW, 1),
                           (NW, 4, BW, 4)).reshape(B * 16, 1)
    return _tc_dense(t1r, h1r, thr, t2c, h2c,
                     W1, b1.reshape(1, HD), W2, b2.reshape(1, NT))
